# baseline (speedup 1.0000x reference)
import functools

import jax
import jax.numpy as jnp
from jax import lax
from jax.experimental import pallas as pl
from jax.experimental.pallas import tpu as pltpu
from jax.experimental.pallas import tpu_sc as plsc

D = 64
NC, NS = 2, 16
NW = NC * NS
CHUNK = 128


def _emb_body(idx_hbm, table_hbm, out_hbm, idx_v, rows_v, sem):
    wid = lax.axis_index("s") * NC + lax.axis_index("c")
    nch = idx_v.shape[0]
    base = wid * (nch * CHUNK)
    pltpu.sync_copy(idx_hbm.at[wid], idx_v)

    def body(j, carry):
        pltpu.async_copy(table_hbm.at[idx_v.at[j]], rows_v, sem).wait()
        pltpu.sync_copy(rows_v, out_hbm.at[pl.ds(base + j * CHUNK, CHUNK)])
        return carry

    lax.fori_loop(0, nch, body, 0)


@functools.partial(jax.jit)
def kernel(token_ids, weight):
    bt, lt = token_ids.shape
    n = bt * lt
    nch = n // (NW * CHUNK)
    idx = token_ids.reshape(NW, nch, CHUNK).astype(jnp.int32)
    mesh = plsc.VectorSubcoreMesh(core_axis_name="c", subcore_axis_name="s")
    run = pl.kernel(
        _emb_body,
        mesh=mesh,
        out_type=jax.ShapeDtypeStruct((n, D), jnp.float32),
        scratch_types=[
            pltpu.VMEM((nch, CHUNK), jnp.int32),
            pltpu.VMEM((CHUNK, D), jnp.float32),
            pltpu.SemaphoreType.DMA,
        ],
        compiler_params=pltpu.CompilerParams(use_tc_tiling_on_sc=False),
    )
    out = run(idx, weight)
    return out.reshape(bt, lt, D)


# speedup vs baseline: 1.1178x; 1.1178x over previous
import functools

import jax
import jax.numpy as jnp
from jax import lax
from jax.experimental import pallas as pl
from jax.experimental.pallas import tpu as pltpu
from jax.experimental.pallas import tpu_sc as plsc

D = 64
NC, NS = 2, 16
NW = NC * NS
CHUNK = 128
NBUF = 8
K = 4


def _emb_body(idx_hbm, table_hbm, out_hbm, idx_v, rows_v, gsem, ssem):
    wid = lax.axis_index("s") * NC + lax.axis_index("c")
    nch = idx_v.shape[0]
    ngrp = nch // NBUF
    base = wid * (nch * CHUNK)
    pltpu.sync_copy(idx_hbm.at[wid], idx_v)

    def gfire(j, slot):
        pltpu.async_copy(table_hbm.at[idx_v.at[j]], rows_v.at[slot],
                         gsem.at[slot])

    def gwait(j, slot):
        pltpu.make_async_copy(table_hbm.at[idx_v.at[j]], rows_v.at[slot],
                              gsem.at[slot]).wait()

    def sfire(j, slot):
        pltpu.async_copy(rows_v.at[slot],
                         out_hbm.at[pl.ds(base + j * CHUNK, CHUNK)],
                         ssem.at[slot])

    def swait(j, slot):
        pltpu.make_async_copy(rows_v.at[slot],
                              out_hbm.at[pl.ds(base + j * CHUNK, CHUNK)],
                              ssem.at[slot]).wait()

    for b in range(K):
        gfire(b, b)

    for b in range(NBUF):
        gwait(b, b)
        sfire(b, b)
        f = b + K
        if f < NBUF:
            gfire(f, f)
        else:
            swait(f - NBUF, f - NBUF)
            gfire(f, f - NBUF)

    def group(g, carry):
        i0 = g * NBUF
        for b in range(NBUF):
            i = i0 + b
            s = (b + K) % NBUF
            gwait(i, b)
            sfire(i, b)
            swait(i + K - NBUF, s)
            gfire(i + K, s)
        return carry

    lax.fori_loop(1, ngrp - 1, group, 0)

    i0 = (ngrp - 1) * NBUF
    for b in range(NBUF):
        i = i0 + b
        gwait(i, b)
        sfire(i, b)
        if b + K < NBUF:
            s = (b + K) % NBUF
            swait(i + K - NBUF, s)
            gfire(i + K, s)

    for b in range(NBUF):
        swait(i0 + b, b)


@functools.partial(jax.jit)
def kernel(token_ids, weight):
    bt, lt = token_ids.shape
    n = bt * lt
    nch = n // (NW * CHUNK)
    idx = token_ids.reshape(NW, nch, CHUNK).astype(jnp.int32)
    mesh = plsc.VectorSubcoreMesh(core_axis_name="c", subcore_axis_name="s")
    run = pl.kernel(
        _emb_body,
        mesh=mesh,
        out_type=jax.ShapeDtypeStruct((n, D), jnp.float32),
        scratch_types=[
            pltpu.VMEM((nch, CHUNK), jnp.int32),
            pltpu.VMEM((NBUF, CHUNK, D), jnp.float32),
            pltpu.SemaphoreType.DMA((NBUF,)),
            pltpu.SemaphoreType.DMA((NBUF,)),
        ],
        compiler_params=pltpu.CompilerParams(use_tc_tiling_on_sc=False),
    )
    out = run(idx, weight)
    return out.reshape(bt, lt, D)
